# baseline (speedup 1.0000x reference)
import functools

import jax
import jax.numpy as jnp
from jax import lax
from jax.experimental import pallas as pl
from jax.experimental.pallas import tpu as pltpu
from jax.experimental.pallas import tpu_sc as plsc

K = 8192
D = 256
N = 8192
R = 1024
NB = N // R
KC = 2048
NKC = K // KC
ST = 2
RS = R // ST
BETA = 0.25
GAMMA = 0.1
SCALE = 200.0
LOG2E = 1.4426950408889634
LN2 = 0.6931471805599453
S2 = SCALE * LOG2E

_SC_CORES = 2
_SC_SUBCORES = 16
_NW = _SC_CORES * _SC_SUBCORES
_BPW = N // _NW


def _tc_body(z_ref, w_ref, idx_ref, stats_ref, d_ref, mhist_ref, colsum_ref,
             counts_ref):
    i = pl.program_id(0)
    zb = z_ref[...]

    @pl.when(i == 0)
    def _init():
        colsum_ref[...] = jnp.zeros((NKC, 1, KC), jnp.float32)
        counts_ref[...] = jnp.zeros((NKC, 1, KC), jnp.float32)
        stats_ref[...] = jnp.zeros((8, 128), jnp.float32)

    lane_f = lax.broadcasted_iota(jnp.int32, (1, KC), 1).astype(jnp.float32)

    def strip(j, wj, s, carry):
        zs = zb[s * RS:(s + 1) * RS, :]
        dj = lax.dot_general(zs, wj, (((1,), (1,)), ((), ())),
                             preferred_element_type=jnp.float32)
        cm = jnp.max(dj, axis=1, keepdims=True)
        local = jnp.min(jnp.where(dj == cm, lane_f, float(KC)), axis=1,
                        keepdims=True)
        cand = local + jnp.float32(KC) * j
        if carry is None:
            m_new = cm
            idx_new = cand
            sml_new = S2 * cm
            t = S2 * dj - sml_new
            e = jnp.exp2(t)
            zsum_new = jnp.sum(e, axis=1, keepdims=True)
            ssum_new = jnp.sum(t * e, axis=1, keepdims=True)
        else:
            m, sml, zsum, ssum, idx = carry
            m_new = jnp.maximum(m, cm)
            sml_new = S2 * m_new
            c = jnp.exp2(sml - sml_new)
            t = S2 * dj - sml_new
            e = jnp.exp2(t)
            zsum_new = zsum * c + jnp.sum(e, axis=1, keepdims=True)
            ssum_new = (ssum + (sml - sml_new) * zsum) * c \
                + jnp.sum(t * e, axis=1, keepdims=True)
            idx_new = jnp.where(cm > m, cand,
                                jnp.where(cm == m, jnp.minimum(idx, cand),
                                          idx))
        d_ref[j, s * RS:(s + 1) * RS, :] = e.astype(jnp.bfloat16)
        mhist_ref[j, s * RS:(s + 1) * RS, :] = sml_new
        return m_new, sml_new, zsum_new, ssum_new, idx_new

    def chunk(j, carry):
        wj = w_ref[j]
        subs = [strip(j, wj, s,
                      None if carry is None else
                      tuple(v[s * RS:(s + 1) * RS, :] for v in carry))
                for s in range(ST)]
        return tuple(jnp.concatenate(parts, axis=0) for parts in zip(*subs))

    carry0 = chunk(0, None)
    m, sml, zsum, ssum, idx = lax.fori_loop(
        1, NKC, lambda j, carry: chunk(j, carry), carry0)

    rz = 1.0 / zsum
    ones8 = jnp.ones((8, R), jnp.bfloat16)

    def passB(j, carry):
        f = jnp.exp2(mhist_ref[j] - sml) * rz
        f8 = jnp.broadcast_to(jnp.reshape(f, (1, R)),
                              (8, R)).astype(jnp.bfloat16)
        csum = lax.dot_general(f8, d_ref[j], (((1,), (0,)), ((), ())),
                               preferred_element_type=jnp.float32)
        colsum_ref[j] = colsum_ref[j] + csum[0:1, :]
        idx_local = idx - jnp.float32(KC) * j
        onehot = (lane_f == idx_local).astype(jnp.bfloat16)
        ccnt = lax.dot_general(ones8, onehot, (((1,), (0,)), ((), ())),
                               preferred_element_type=jnp.float32)
        counts_ref[j] = counts_ref[j] + ccnt[0:1, :]
        return carry

    lax.fori_loop(0, NKC, passB, 0)

    idx_ref[...] = idx.astype(jnp.int32).reshape(1, R, 1)

    plogp = LN2 * ssum / zsum - jnp.log(zsum)
    row = lax.broadcasted_iota(jnp.int32, (8, 128), 0)
    contrib = (jnp.where(row == 0, jnp.sum(plogp), 0.0)
               + jnp.where(row == 1, jnp.sum(m), 0.0)
               + jnp.where(row == 2, jnp.sum(zb * zb), 0.0))
    stats_ref[...] += contrib.astype(jnp.float32)

    @pl.when(i == NB - 1)
    def _finish():
        ones8 = jnp.full((8, D), 0.125, jnp.float32)

        def fin(j, carry):
            wdot, entc, enta = carry
            wj = w_ref[j]
            wn8 = lax.dot_general(ones8, wj * wj, (((1,), (1,)), ((), ())),
                                  preferred_element_type=jnp.float32)
            cj = counts_ref[j]
            wdot = wdot + jnp.sum(cj * wn8[0:1, :] * 8.0)
            probs = cj * (1.0 / N)
            entc = entc + jnp.sum(probs * jnp.log(probs + 1e-10))
            avgp = colsum_ref[j] * (1.0 / N)
            enta = enta + jnp.sum(avgp * jnp.log(avgp + 1e-10))
            return wdot, entc, enta

        wdot, entc, enta = lax.fori_loop(0, NKC, fin, (0.0, 0.0, 0.0))
        stats_ref[...] += (jnp.where(row == 3, wdot, 0.0)
                           + jnp.where(row == 4, entc, 0.0)
                           + jnp.where(row == 5, enta, 0.0)).astype(jnp.float32)


def _tc_stats(z2, w3):
    return pl.pallas_call(
        _tc_body,
        grid=(NB,),
        in_specs=[
            pl.BlockSpec((R, D), lambda i: (i, 0)),
            pl.BlockSpec((NKC, KC, D), lambda i: (0, 0, 0)),
        ],
        out_specs=[
            pl.BlockSpec((1, R, 1), lambda i: (i, 0, 0)),
            pl.BlockSpec((8, 128), lambda i: (0, 0)),
        ],
        out_shape=[
            jax.ShapeDtypeStruct((NB, R, 1), jnp.int32),
            jax.ShapeDtypeStruct((8, 128), jnp.float32),
        ],
        scratch_shapes=[
            pltpu.VMEM((NKC, R, KC), jnp.bfloat16),
            pltpu.VMEM((NKC, R, 1), jnp.float32),
            pltpu.VMEM((NKC, 1, KC), jnp.float32),
            pltpu.VMEM((NKC, 1, KC), jnp.float32),
        ],
    )(z2, w3)


def _gather_rows(table, idx_flat):
    mesh = plsc.VectorSubcoreMesh(core_axis_name="c", subcore_axis_name="s")

    @functools.partial(
        pl.kernel,
        mesh=mesh,
        out_type=jax.ShapeDtypeStruct((N, D), jnp.float32),
        scratch_types=[
            pltpu.VMEM((_BPW,), jnp.int32),
            pltpu.VMEM((_BPW, D), jnp.float32),
            pltpu.SemaphoreType.DMA,
        ],
    )
    def gk(table_hbm, idx_hbm, out_hbm, idx_v, rows_v, sem):
        wid = lax.axis_index("s") * _SC_CORES + lax.axis_index("c")
        base = wid * _BPW
        pltpu.sync_copy(idx_hbm.at[pl.ds(base, _BPW)], idx_v)
        pltpu.async_copy(table_hbm.at[idx_v], rows_v, sem).wait()
        pltpu.sync_copy(rows_v, out_hbm.at[pl.ds(base, _BPW)])

    return gk(table, idx_flat)


def kernel(z, W):
    z2 = z.reshape(N, D)
    w3 = W.reshape(NKC, KC, D)
    idx3, stats = _tc_stats(z2, w3)
    idx_flat = idx3.reshape(N)
    z_q = _gather_rows(W, idx_flat)

    plogp_total = stats[0, 0]
    dmax_sum = stats[1, 0]
    znorm_sum = stats[2, 0]
    wdot = stats[3, 0]
    entc = stats[4, 0]
    enta = stats[5, 0]

    mse = (znorm_sum - 2.0 * dmax_sum + wdot) / (N * D)
    sample_entropy = -plogp_total / N
    avg_entropy = -enta
    loss = (1.0 + BETA) * mse + GAMMA * (sample_entropy - avg_entropy)
    perplexity = jnp.exp(-entc)

    z_q_out = z_q.reshape(z.shape)
    embed_indices = idx_flat.reshape(z.shape[0], z.shape[1])
    return (z_q_out, embed_indices, loss, perplexity)


# speedup vs baseline: 1.1254x; 1.1254x over previous
import functools

import jax
import jax.numpy as jnp
from jax import lax
from jax.experimental import pallas as pl
from jax.experimental.pallas import tpu as pltpu
from jax.experimental.pallas import tpu_sc as plsc

K = 8192
D = 256
N = 8192
R = 1024
NB = N // R
KC = 4096
NKC = K // KC
ST = 4
RS = R // ST
BETA = 0.25
GAMMA = 0.1
SCALE = 200.0
LOG2E = 1.4426950408889634
LN2 = 0.6931471805599453
S2 = SCALE * LOG2E

_SC_CORES = 2
_SC_SUBCORES = 16
_NW = _SC_CORES * _SC_SUBCORES
_BPW = N // _NW


def _tc_body(z_ref, w_ref, idx_ref, stats_ref, d_ref, mhist_ref, colsum_ref,
             counts_ref):
    i = pl.program_id(0)
    zb = z_ref[...]

    @pl.when(i == 0)
    def _init():
        colsum_ref[...] = jnp.zeros((NKC, 1, KC), jnp.float32)
        counts_ref[...] = jnp.zeros((NKC, 1, KC), jnp.float32)
        stats_ref[...] = jnp.zeros((8, 128), jnp.float32)

    lane_f = lax.broadcasted_iota(jnp.int32, (1, KC), 1).astype(jnp.float32)

    def strip(j, wj, s, carry):
        zs = zb[s * RS:(s + 1) * RS, :]
        dj = lax.dot_general(zs, wj, (((1,), (1,)), ((), ())),
                             preferred_element_type=jnp.float32)
        cm = jnp.max(dj, axis=1, keepdims=True)
        local = jnp.min(jnp.where(dj == cm, lane_f, float(KC)), axis=1,
                        keepdims=True)
        cand = local + jnp.float32(KC) * j
        if carry is None:
            m_new = cm
            idx_new = cand
            sml_new = S2 * cm
            t = S2 * dj - sml_new
            e = jnp.exp2(t)
            zsum_new = jnp.sum(e, axis=1, keepdims=True)
            ssum_new = jnp.sum(t * e, axis=1, keepdims=True)
        else:
            m, sml, zsum, ssum, idx = carry
            m_new = jnp.maximum(m, cm)
            sml_new = S2 * m_new
            c = jnp.exp2(sml - sml_new)
            t = S2 * dj - sml_new
            e = jnp.exp2(t)
            zsum_new = zsum * c + jnp.sum(e, axis=1, keepdims=True)
            ssum_new = (ssum + (sml - sml_new) * zsum) * c \
                + jnp.sum(t * e, axis=1, keepdims=True)
            idx_new = jnp.where(cm > m, cand,
                                jnp.where(cm == m, jnp.minimum(idx, cand),
                                          idx))
        d_ref[j, s * RS:(s + 1) * RS, :] = e.astype(jnp.bfloat16)
        mhist_ref[j, s * RS:(s + 1) * RS, :] = sml_new
        return m_new, sml_new, zsum_new, ssum_new, idx_new

    def chunk(j, carry):
        wj = w_ref[j]
        subs = [strip(j, wj, s,
                      None if carry is None else
                      tuple(v[s * RS:(s + 1) * RS, :] for v in carry))
                for s in range(ST)]
        return tuple(jnp.concatenate(parts, axis=0) for parts in zip(*subs))

    carry0 = chunk(0, None)
    m, sml, zsum, ssum, idx = lax.fori_loop(
        1, NKC, lambda j, carry: chunk(j, carry), carry0)

    rz = 1.0 / zsum
    ones8 = jnp.ones((8, R), jnp.bfloat16)

    def passB(j, carry):
        f = jnp.exp2(mhist_ref[j] - sml) * rz
        f8 = jnp.broadcast_to(jnp.reshape(f, (1, R)),
                              (8, R)).astype(jnp.bfloat16)
        csum = lax.dot_general(f8, d_ref[j], (((1,), (0,)), ((), ())),
                               preferred_element_type=jnp.float32)
        colsum_ref[j] = colsum_ref[j] + csum[0:1, :]
        idx_local = idx - jnp.float32(KC) * j
        onehot = (lane_f == idx_local).astype(jnp.bfloat16)
        ccnt = lax.dot_general(ones8, onehot, (((1,), (0,)), ((), ())),
                               preferred_element_type=jnp.float32)
        counts_ref[j] = counts_ref[j] + ccnt[0:1, :]
        return carry

    lax.fori_loop(0, NKC, passB, 0)

    idx_ref[...] = idx.astype(jnp.int32).reshape(1, R, 1)

    plogp = LN2 * ssum / zsum - jnp.log(zsum)
    row = lax.broadcasted_iota(jnp.int32, (8, 128), 0)
    contrib = (jnp.where(row == 0, jnp.sum(plogp), 0.0)
               + jnp.where(row == 1, jnp.sum(m), 0.0)
               + jnp.where(row == 2, jnp.sum(zb * zb), 0.0))
    stats_ref[...] += contrib.astype(jnp.float32)

    @pl.when(i == NB - 1)
    def _finish():
        ones8 = jnp.full((8, D), 0.125, jnp.float32)

        def fin(j, carry):
            wdot, entc, enta = carry
            wj = w_ref[j]
            wn8 = lax.dot_general(ones8, wj * wj, (((1,), (1,)), ((), ())),
                                  preferred_element_type=jnp.float32)
            cj = counts_ref[j]
            wdot = wdot + jnp.sum(cj * wn8[0:1, :] * 8.0)
            probs = cj * (1.0 / N)
            entc = entc + jnp.sum(probs * jnp.log(probs + 1e-10))
            avgp = colsum_ref[j] * (1.0 / N)
            enta = enta + jnp.sum(avgp * jnp.log(avgp + 1e-10))
            return wdot, entc, enta

        wdot, entc, enta = lax.fori_loop(0, NKC, fin, (0.0, 0.0, 0.0))
        stats_ref[...] += (jnp.where(row == 3, wdot, 0.0)
                           + jnp.where(row == 4, entc, 0.0)
                           + jnp.where(row == 5, enta, 0.0)).astype(jnp.float32)


def _tc_stats(z2, w3):
    return pl.pallas_call(
        _tc_body,
        grid=(NB,),
        in_specs=[
            pl.BlockSpec((R, D), lambda i: (i, 0)),
            pl.BlockSpec((NKC, KC, D), lambda i: (0, 0, 0)),
        ],
        out_specs=[
            pl.BlockSpec((1, R, 1), lambda i: (i, 0, 0)),
            pl.BlockSpec((8, 128), lambda i: (0, 0)),
        ],
        out_shape=[
            jax.ShapeDtypeStruct((NB, R, 1), jnp.int32),
            jax.ShapeDtypeStruct((8, 128), jnp.float32),
        ],
        scratch_shapes=[
            pltpu.VMEM((NKC, R, KC), jnp.bfloat16),
            pltpu.VMEM((NKC, R, 1), jnp.float32),
            pltpu.VMEM((NKC, 1, KC), jnp.float32),
            pltpu.VMEM((NKC, 1, KC), jnp.float32),
        ],
    )(z2, w3)


def _gather_rows(table, idx_flat):
    mesh = plsc.VectorSubcoreMesh(core_axis_name="c", subcore_axis_name="s")

    @functools.partial(
        pl.kernel,
        mesh=mesh,
        out_type=jax.ShapeDtypeStruct((N, D), jnp.float32),
        scratch_types=[
            pltpu.VMEM((_BPW,), jnp.int32),
            pltpu.VMEM((_BPW, D), jnp.float32),
            pltpu.SemaphoreType.DMA,
        ],
    )
    def gk(table_hbm, idx_hbm, out_hbm, idx_v, rows_v, sem):
        wid = lax.axis_index("s") * _SC_CORES + lax.axis_index("c")
        base = wid * _BPW
        pltpu.sync_copy(idx_hbm.at[pl.ds(base, _BPW)], idx_v)
        pltpu.async_copy(table_hbm.at[idx_v], rows_v, sem).wait()
        pltpu.sync_copy(rows_v, out_hbm.at[pl.ds(base, _BPW)])

    return gk(table, idx_flat)


def kernel(z, W):
    z2 = z.reshape(N, D)
    w3 = W.reshape(NKC, KC, D)
    idx3, stats = _tc_stats(z2, w3)
    idx_flat = idx3.reshape(N)
    z_q = _gather_rows(W, idx_flat)

    plogp_total = stats[0, 0]
    dmax_sum = stats[1, 0]
    znorm_sum = stats[2, 0]
    wdot = stats[3, 0]
    entc = stats[4, 0]
    enta = stats[5, 0]

    mse = (znorm_sum - 2.0 * dmax_sum + wdot) / (N * D)
    sample_entropy = -plogp_total / N
    avg_entropy = -enta
    loss = (1.0 + BETA) * mse + GAMMA * (sample_entropy - avg_entropy)
    perplexity = jnp.exp(-entc)

    z_q_out = z_q.reshape(z.shape)
    embed_indices = idx_flat.reshape(z.shape[0], z.shape[1])
    return (z_q_out, embed_indices, loss, perplexity)
